# baseline (speedup 1.0000x reference)
import jax
import jax.numpy as jnp
from jax.experimental import pallas as pl


def _copy_kernel(w_ref, out_ref):
    out_ref[0] = w_ref[...]


def kernel(tokens, W_pos):
    batch = tokens.shape[0]
    seq_len = tokens.shape[1]
    d_model = W_pos.shape[1]

    chunk = 512
    if seq_len % chunk != 0:
        chunk = seq_len
    n_chunks = seq_len // chunk

    return pl.pallas_call(
        _copy_kernel,
        grid=(n_chunks, batch),
        in_specs=[
            pl.BlockSpec((chunk, d_model), lambda i, j: (i, 0)),
        ],
        out_specs=pl.BlockSpec((1, chunk, d_model), lambda i, j: (j, i, 0)),
        out_shape=jax.ShapeDtypeStruct((batch, seq_len, d_model), W_pos.dtype),
    )(W_pos[:seq_len])


# speedup vs baseline: 1.4561x; 1.4561x over previous
import jax
import jax.numpy as jnp
from jax.experimental import pallas as pl
from jax.experimental.pallas import tpu as pltpu


def _bcast_kernel(w_hbm, out_hbm, w_vmem, in_sem, out_sems):
    batch = out_hbm.shape[0]
    load = pltpu.make_async_copy(w_hbm, w_vmem, in_sem)
    load.start()
    load.wait()
    copies = [
        pltpu.make_async_copy(w_vmem, out_hbm.at[j], out_sems.at[j])
        for j in range(batch)
    ]
    for c in copies:
        c.start()
    for c in copies:
        c.wait()


def kernel(tokens, W_pos):
    batch = tokens.shape[0]
    seq_len = tokens.shape[1]
    d_model = W_pos.shape[1]

    return pl.pallas_call(
        _bcast_kernel,
        in_specs=[pl.BlockSpec(memory_space=pl.ANY)],
        out_specs=pl.BlockSpec(memory_space=pl.ANY),
        out_shape=jax.ShapeDtypeStruct((batch, seq_len, d_model), W_pos.dtype),
        scratch_shapes=[
            pltpu.VMEM((seq_len, d_model), W_pos.dtype),
            pltpu.SemaphoreType.DMA,
            pltpu.SemaphoreType.DMA((batch,)),
        ],
    )(W_pos[:seq_len])


# speedup vs baseline: 1.5545x; 1.0676x over previous
import jax
import jax.numpy as jnp
from jax.experimental import pallas as pl
from jax.experimental.pallas import tpu as pltpu

_N_CHUNKS = 4


def _bcast_kernel(w_hbm, out_hbm, w_vmem, in_sems, out_sems):
    batch = out_hbm.shape[0]
    seq_len = w_hbm.shape[0]
    chunk = seq_len // _N_CHUNKS

    loads = [
        pltpu.make_async_copy(
            w_hbm.at[pl.ds(i * chunk, chunk)],
            w_vmem.at[pl.ds(i * chunk, chunk)],
            in_sems.at[i],
        )
        for i in range(_N_CHUNKS)
    ]
    for ld in loads:
        ld.start()

    stores = []
    for i in range(_N_CHUNKS):
        loads[i].wait()
        for j in range(batch):
            st = pltpu.make_async_copy(
                w_vmem.at[pl.ds(i * chunk, chunk)],
                out_hbm.at[j, pl.ds(i * chunk, chunk)],
                out_sems.at[j],
            )
            st.start()
            stores.append(st)
    for st in stores:
        st.wait()


def kernel(tokens, W_pos):
    batch = tokens.shape[0]
    seq_len = tokens.shape[1]
    d_model = W_pos.shape[1]

    return pl.pallas_call(
        _bcast_kernel,
        in_specs=[pl.BlockSpec(memory_space=pl.ANY)],
        out_specs=pl.BlockSpec(memory_space=pl.ANY),
        out_shape=jax.ShapeDtypeStruct((batch, seq_len, d_model), W_pos.dtype),
        scratch_shapes=[
            pltpu.VMEM((seq_len, d_model), W_pos.dtype),
            pltpu.SemaphoreType.DMA((_N_CHUNKS,)),
            pltpu.SemaphoreType.DMA((batch,)),
        ],
    )(W_pos[:seq_len])


# speedup vs baseline: 1.5600x; 1.0036x over previous
import jax
import jax.numpy as jnp
from jax.experimental import pallas as pl
from jax.experimental.pallas import tpu as pltpu

_N_CHUNKS = 8


def _bcast_kernel(w_hbm, out_hbm, w_vmem, in_sems, out_sems):
    batch = out_hbm.shape[0]
    seq_len = w_hbm.shape[0]
    chunk = seq_len // _N_CHUNKS

    loads = [
        pltpu.make_async_copy(
            w_hbm.at[pl.ds(i * chunk, chunk)],
            w_vmem.at[pl.ds(i * chunk, chunk)],
            in_sems.at[i],
        )
        for i in range(_N_CHUNKS)
    ]
    for ld in loads:
        ld.start()

    stores = []
    for i in range(_N_CHUNKS):
        loads[i].wait()
        for j in range(batch):
            st = pltpu.make_async_copy(
                w_vmem.at[pl.ds(i * chunk, chunk)],
                out_hbm.at[j, pl.ds(i * chunk, chunk)],
                out_sems.at[j],
            )
            st.start()
            stores.append(st)
    for st in stores:
        st.wait()


def kernel(tokens, W_pos):
    batch = tokens.shape[0]
    seq_len = tokens.shape[1]
    d_model = W_pos.shape[1]

    return pl.pallas_call(
        _bcast_kernel,
        in_specs=[pl.BlockSpec(memory_space=pl.ANY)],
        out_specs=pl.BlockSpec(memory_space=pl.ANY),
        out_shape=jax.ShapeDtypeStruct((batch, seq_len, d_model), W_pos.dtype),
        scratch_shapes=[
            pltpu.VMEM((seq_len, d_model), W_pos.dtype),
            pltpu.SemaphoreType.DMA((_N_CHUNKS,)),
            pltpu.SemaphoreType.DMA((batch,)),
        ],
    )(W_pos[:seq_len])
